# baseline (speedup 1.0000x reference)
import functools

import jax
import jax.numpy as jnp
from jax.experimental import pallas as pl
from jax.experimental.pallas import tpu as pltpu

N = 5000
NP = 5120
B = 512
NB = NP // B
MAXD = 100
SCORE_THRESH = 0.05
NMS_THRESH = 0.5


def _body(draw_ref, out_ref, rawr_ref, ds_ref, srows_ref, keep_ref,
          dcat_ref):
    f32 = jnp.float32
    i32 = jnp.int32
    io0 = jax.lax.broadcasted_iota(i32, (B, 1), 0).astype(f32)
    io1 = jax.lax.broadcasted_iota(i32, (1, B), 1).astype(f32)
    ii = jax.lax.broadcasted_iota(i32, (B, B), 0)
    jj = jax.lax.broadcasted_iota(i32, (B, B), 1)
    ult = (ii <= jj).astype(f32)
    colpos = jax.lax.broadcasted_iota(i32, (1, NP), 1).astype(f32)

    def tcol2row(v):
        return jnp.transpose(jnp.broadcast_to(v, (B, 8)), (1, 0))[0:1, :]

    def trow2col(v):
        return jnp.transpose(jnp.broadcast_to(v, (8, B)), (1, 0))[:, 0:1]

    def split3(x):
        xi = jax.lax.bitcast_convert_type(x, jnp.int32)
        hi = jax.lax.bitcast_convert_type(
            xi & jnp.int32(-65536), f32)
        r1 = x - hi
        ri = jax.lax.bitcast_convert_type(r1, jnp.int32)
        mid = jax.lax.bitcast_convert_type(
            ri & jnp.int32(-65536), f32)
        lo = r1 - mid
        return hi, mid, lo

    def p0(ib, _):
        c0 = pl.multiple_of(ib * B, B)
        blk = draw_ref[pl.ds(c0, B), :]
        sc = blk[:, 4:5]
        st = jnp.where(sc > SCORE_THRESH, sc, -1.0)
        rawr_ref[4:5, pl.ds(c0, B)] = tcol2row(st)
        dth = jnp.concatenate(
            [blk[:, 0:4], st, jnp.zeros((B, 3), f32)], axis=1)
        hi, mid, lo = split3(dth)
        dcat_ref[pl.ds(c0, B), 0:8] = hi.astype(jnp.bfloat16)
        dcat_ref[pl.ds(c0, B), 8:16] = mid.astype(jnp.bfloat16)
        dcat_ref[pl.ds(c0, B), 16:24] = lo.astype(jnp.bfloat16)
        return 0

    jax.lax.fori_loop(0, NB, p0, 0)

    sr = rawr_ref[4:5, :]
    ones_np = jnp.ones((NP, 8), jnp.bfloat16)
    dn = (((1,), (0,)), ((), ()))
    def p1(ib, _):
        c0 = pl.multiple_of(ib * B, B)
        sc = draw_ref[pl.ds(c0, B), 4:5]
        st = jnp.where(sc > SCORE_THRESH, sc, -1.0)
        m = ((sr > st)
             | ((sr == st) & (colpos < (io0 + ib * B))))
        rank = jax.lax.dot_general(m.astype(jnp.bfloat16), ones_np, dn,
                                   preferred_element_type=f32)[:, 0:1]
        rawr_ref[5:6, pl.ds(c0, B)] = tcol2row(rank)
        return 0

    jax.lax.fori_loop(0, NB, p1, 0)

    rankrow = rawr_ref[5:6, :]

    def p2(ob, _):
        c0 = pl.multiple_of(ob * B, B)
        oh = (rankrow == (io0 + ob * B)).astype(jnp.bfloat16)
        m = jax.lax.dot_general(oh, dcat_ref[:, :], dn,
                                preferred_element_type=f32)
        dsb = m[:, 0:8] + m[:, 8:16] + m[:, 16:24]
        ds_ref[pl.ds(c0, B), :] = dsb
        for c in range(5):
            srows_ref[c:c + 1, pl.ds(c0, B)] = tcol2row(dsb[:, c:c + 1])
        acol = (dsb[:, 2:3] - dsb[:, 0:1]) * (dsb[:, 3:4] - dsb[:, 1:2])
        srows_ref[5:6, pl.ds(c0, B)] = tcol2row(acol)
        return 0

    jax.lax.fori_loop(0, NB, p2, 0)

    ssr = srows_ref[4:5, :]
    keep_ref[0:1, :] = (ssr > -0.5).astype(f32)

    def iou_cols(x1c, y1c, x2c, y2c, ac, xr1, yr1, xr2, yr2, arow):
        ltx = jnp.maximum(x1c, xr1)
        lty = jnp.maximum(y1c, yr1)
        rbx = jnp.minimum(x2c, xr2)
        rby = jnp.minimum(y2c, yr2)
        wx = jnp.maximum(rbx - ltx, 0.0)
        wy = jnp.maximum(rby - lty, 0.0)
        inter = wx * wy
        return inter / ((ac + arow) - inter + 1e-7)

    W = 1280
    NCH = NP // W
    iow = jax.lax.broadcasted_iota(jnp.int32, (1, W), 1)

    def p3(ib, _):
        c0 = pl.multiple_of(ib * B, B)
        x1c = ds_ref[pl.ds(c0, B), 0:1]
        y1c = ds_ref[pl.ds(c0, B), 1:2]
        x2c = ds_ref[pl.ds(c0, B), 2:3]
        y2c = ds_ref[pl.ds(c0, B), 3:4]
        ac = (x2c - x1c) * (y2c - y1c)
        xb1 = srows_ref[0:1, pl.ds(c0, B)]
        yb1 = srows_ref[1:2, pl.ds(c0, B)]
        xb2 = srows_ref[2:3, pl.ds(c0, B)]
        yb2 = srows_ref[3:4, pl.ds(c0, B)]
        ab = srows_ref[5:6, pl.ds(c0, B)]
        ovs = iou_cols(x1c, y1c, x2c, y2c, ac, xb1, yb1, xb2, yb2, ab) \
            > NMS_THRESH
        amat = (ovs & (jj < ii)).astype(f32)
        kb0 = trow2col(keep_ref[0:1, pl.ds(c0, B)])

        def fcond(st):
            return st[1] & (st[2] < B + 4)

        def step(kb):
            sup = jax.lax.dot_general(amat, kb, (((1,), (0,)), ((), ())),
                                      preferred_element_type=f32)
            return kb0 * jnp.where(sup > 0, 0.0, 1.0)

        def fbody(st):
            kb = step(step(st[0]))
            changed = jnp.sum(jnp.abs(kb - st[0])) > 0
            return (kb, changed, st[2] + 1)

        kb = jax.lax.while_loop(fcond, fbody,
                                (kb0, jnp.bool_(True), jnp.int32(0)))[0]
        krow = tcol2row(kb)
        keep_ref[0:1, pl.ds(c0, B)] = krow

        def sweep(ch, _):
            j0 = pl.multiple_of(ch * W, W)
            xr1 = srows_ref[0:1, pl.ds(j0, W)]
            yr1 = srows_ref[1:2, pl.ds(j0, W)]
            xr2 = srows_ref[2:3, pl.ds(j0, W)]
            yr2 = srows_ref[3:4, pl.ds(j0, W)]
            arw = srows_ref[5:6, pl.ds(j0, W)]
            ovf = (iou_cols(x1c, y1c, x2c, y2c, ac, xr1, yr1, xr2, yr2, arw)
                   > NMS_THRESH).astype(f32)
            supcnt = jax.lax.dot_general(krow, ovf, (((1,), (0,)), ((), ())),
                                         preferred_element_type=f32)
            later = (iow + j0) >= (c0 + B)
            kch = keep_ref[0:1, pl.ds(j0, W)]
            keep_ref[0:1, pl.ds(j0, W)] = \
                kch * jnp.where((supcnt > 0) & later, 0.0, 1.0)
            return 0

        jstart = (c0 + B) // W
        jax.lax.fori_loop(jstart, NCH, sweep, 0)
        return 0

    jax.lax.fori_loop(0, NB, p3, 0)

    kfull = keep_ref[0:1, :]
    ktot = jnp.sum(kfull)
    out_ref[:, :] = jnp.zeros((B, 8), f32)

    def p4(ib, off):
        c0 = pl.multiple_of(ib * B, B)
        krow = keep_ref[0:1, pl.ds(c0, B)]
        prefix = jax.lax.dot_general(krow, ult, (((1,), (0,)), ((), ())),
                                     preferred_element_type=f32)
        cum = off + prefix
        posg = io1 + ib * B
        rnk = jnp.where(krow > 0, cum - 1.0, ktot + posg - cum)
        oh = (rnk == io0).astype(jnp.bfloat16)
        dsb = ds_ref[pl.ds(c0, B), :]
        hi, mid, lo = split3(dsb)
        out_ref[:, :] += (
            jax.lax.dot_general(oh, hi.astype(jnp.bfloat16), dn,
                                preferred_element_type=f32)
            + jax.lax.dot_general(oh, mid.astype(jnp.bfloat16), dn,
                                  preferred_element_type=f32)
            + jax.lax.dot_general(oh, lo.astype(jnp.bfloat16), dn,
                                  preferred_element_type=f32))
        return off + jnp.sum(krow)

    jax.lax.fori_loop(0, NB, p4, jnp.float32(0.0))
    sc = out_ref[:, 4:5]
    out_ref[:, 4:5] = jnp.where(io0 < ktot, sc, -1.0)


@functools.partial(jax.jit, static_argnames=("interpret",))
def _run(draw, interpret=False):
    return pl.pallas_call(
        _body,
        out_shape=jax.ShapeDtypeStruct((B, 8), jnp.float32),
        scratch_shapes=[
            pltpu.VMEM((8, NP), jnp.float32),
            pltpu.VMEM((NP, 8), jnp.float32),
            pltpu.VMEM((8, NP), jnp.float32),
            pltpu.VMEM((1, NP), jnp.float32),
            pltpu.VMEM((NP, 24), jnp.bfloat16),
        ],
        interpret=interpret,
    )(draw)


def kernel(boxes, scores, interpret=False):
    bp = jnp.zeros((NP, 4), jnp.float32).at[:N].set(boxes)
    sp = jnp.zeros((NP, 1), jnp.float32).at[:N, 0].set(scores)
    draw = jnp.concatenate([bp, sp, jnp.zeros((NP, 3), jnp.float32)], axis=1)
    out = _run(draw, interpret=interpret)
    return out[:MAXD, :5]


# speedup vs baseline: 1.1466x; 1.1466x over previous
import functools

import jax
import jax.numpy as jnp
from jax.experimental import pallas as pl
from jax.experimental.pallas import tpu as pltpu

N = 5000
NP = 5120
B = 512
NB = NP // B
MAXD = 100
SCORE_THRESH = 0.05
NMS_THRESH = 0.5


def _body(draw_ref, out_ref, rawr_ref, ds_ref, srows_ref, keep_ref,
          dcat_ref):
    f32 = jnp.float32
    i32 = jnp.int32
    io0 = jax.lax.broadcasted_iota(i32, (B, 1), 0).astype(f32)
    io1 = jax.lax.broadcasted_iota(i32, (1, B), 1).astype(f32)
    ii = jax.lax.broadcasted_iota(i32, (B, B), 0)
    jj = jax.lax.broadcasted_iota(i32, (B, B), 1)
    ult = (ii <= jj).astype(f32)
    colpos = jax.lax.broadcasted_iota(i32, (1, NP), 1).astype(f32)

    def tcol2row(v):
        return jnp.transpose(jnp.broadcast_to(v, (B, 8)), (1, 0))[0:1, :]

    def trow2col(v):
        return jnp.transpose(jnp.broadcast_to(v, (8, B)), (1, 0))[:, 0:1]

    def split3(x):
        xi = jax.lax.bitcast_convert_type(x, jnp.int32)
        hi = jax.lax.bitcast_convert_type(
            xi & jnp.int32(-65536), f32)
        r1 = x - hi
        ri = jax.lax.bitcast_convert_type(r1, jnp.int32)
        mid = jax.lax.bitcast_convert_type(
            ri & jnp.int32(-65536), f32)
        lo = r1 - mid
        return hi, mid, lo

    def p0(ib, _):
        c0 = pl.multiple_of(ib * B, B)
        blk = draw_ref[pl.ds(c0, B), :]
        sc = blk[:, 4:5]
        st = jnp.where(sc > SCORE_THRESH, sc, -1.0)
        rawr_ref[4:5, pl.ds(c0, B)] = tcol2row(st)
        dth = jnp.concatenate(
            [blk[:, 0:4], st, jnp.zeros((B, 3), f32)], axis=1)
        hi, mid, lo = split3(dth)
        dcat_ref[pl.ds(c0, B), 0:8] = hi.astype(jnp.bfloat16)
        dcat_ref[pl.ds(c0, B), 8:16] = mid.astype(jnp.bfloat16)
        dcat_ref[pl.ds(c0, B), 16:24] = lo.astype(jnp.bfloat16)
        return 0

    jax.lax.fori_loop(0, NB, p0, 0)

    sr = rawr_ref[4:5, :]
    dn = (((1,), (0,)), ((), ()))
    def p1(ib, _):
        c0 = pl.multiple_of(ib * B, B)
        sc = draw_ref[pl.ds(c0, B), 4:5]
        st = jnp.where(sc > SCORE_THRESH, sc, -1.0)
        gt = (sr > st).astype(f32)
        eq = ((sr == st) & (colpos < (io0 + ib * B))).astype(f32)
        rank = jnp.sum(gt + eq, axis=1, keepdims=True)
        rawr_ref[5:6, pl.ds(c0, B)] = tcol2row(rank)
        return 0

    jax.lax.fori_loop(0, NB, p1, 0)

    rankrow = rawr_ref[5:6, :]

    def p2(ob, _):
        c0 = pl.multiple_of(ob * B, B)
        oh = (rankrow == (io0 + ob * B)).astype(jnp.bfloat16)
        m = jax.lax.dot_general(oh, dcat_ref[:, :], dn,
                                preferred_element_type=f32)
        dsb = m[:, 0:8] + m[:, 8:16] + m[:, 16:24]
        ds_ref[pl.ds(c0, B), :] = dsb
        for c in range(5):
            srows_ref[c:c + 1, pl.ds(c0, B)] = tcol2row(dsb[:, c:c + 1])
        acol = (dsb[:, 2:3] - dsb[:, 0:1]) * (dsb[:, 3:4] - dsb[:, 1:2])
        srows_ref[5:6, pl.ds(c0, B)] = tcol2row(acol)
        return 0

    jax.lax.fori_loop(0, NB, p2, 0)

    ssr = srows_ref[4:5, :]
    keep_ref[0:1, :] = (ssr > -0.5).astype(f32)

    def iou_cols(x1c, y1c, x2c, y2c, ac, xr1, yr1, xr2, yr2, arow):
        ltx = jnp.maximum(x1c, xr1)
        lty = jnp.maximum(y1c, yr1)
        rbx = jnp.minimum(x2c, xr2)
        rby = jnp.minimum(y2c, yr2)
        wx = jnp.maximum(rbx - ltx, 0.0)
        wy = jnp.maximum(rby - lty, 0.0)
        inter = wx * wy
        return inter / ((ac + arow) - inter + 1e-7)

    W = 640
    NCH = NP // W
    iow = jax.lax.broadcasted_iota(jnp.int32, (1, W), 1)

    def p3(ib, _):
        c0 = pl.multiple_of(ib * B, B)
        x1c = ds_ref[pl.ds(c0, B), 0:1]
        y1c = ds_ref[pl.ds(c0, B), 1:2]
        x2c = ds_ref[pl.ds(c0, B), 2:3]
        y2c = ds_ref[pl.ds(c0, B), 3:4]
        ac = (x2c - x1c) * (y2c - y1c)
        xb1 = srows_ref[0:1, pl.ds(c0, B)]
        yb1 = srows_ref[1:2, pl.ds(c0, B)]
        xb2 = srows_ref[2:3, pl.ds(c0, B)]
        yb2 = srows_ref[3:4, pl.ds(c0, B)]
        ab = srows_ref[5:6, pl.ds(c0, B)]
        ovs = iou_cols(x1c, y1c, x2c, y2c, ac, xb1, yb1, xb2, yb2, ab) \
            > NMS_THRESH
        amat = (ovs & (jj < ii)).astype(f32)
        kb0 = trow2col(keep_ref[0:1, pl.ds(c0, B)])

        def fcond(st):
            return st[1] & (st[2] < B + 4)

        def step(kb):
            sup = jax.lax.dot_general(amat, kb, (((1,), (0,)), ((), ())),
                                      preferred_element_type=f32)
            return kb0 * jnp.where(sup > 0, 0.0, 1.0)

        def fbody(st):
            kb = step(step(st[0]))
            changed = jnp.sum(jnp.abs(kb - st[0])) > 0
            return (kb, changed, st[2] + 1)

        kb = jax.lax.while_loop(fcond, fbody,
                                (kb0, jnp.bool_(True), jnp.int32(0)))[0]
        krow = tcol2row(kb)
        keep_ref[0:1, pl.ds(c0, B)] = krow

        def sweep(ch, _):
            j0 = pl.multiple_of(ch * W, W)
            xr1 = srows_ref[0:1, pl.ds(j0, W)]
            yr1 = srows_ref[1:2, pl.ds(j0, W)]
            xr2 = srows_ref[2:3, pl.ds(j0, W)]
            yr2 = srows_ref[3:4, pl.ds(j0, W)]
            arw = srows_ref[5:6, pl.ds(j0, W)]
            ovf = (iou_cols(x1c, y1c, x2c, y2c, ac, xr1, yr1, xr2, yr2, arw)
                   > NMS_THRESH).astype(f32)
            supcnt = jax.lax.dot_general(krow, ovf, (((1,), (0,)), ((), ())),
                                         preferred_element_type=f32)
            later = (iow + j0) >= (c0 + B)
            kch = keep_ref[0:1, pl.ds(j0, W)]
            keep_ref[0:1, pl.ds(j0, W)] = \
                kch * jnp.where((supcnt > 0) & later, 0.0, 1.0)
            return 0

        jstart = (c0 + B) // W
        jax.lax.fori_loop(jstart, NCH, sweep, 0)
        return 0

    jax.lax.fori_loop(0, NB, p3, 0)

    kfull = keep_ref[0:1, :]
    ktot = jnp.sum(kfull)
    out_ref[:, :] = jnp.zeros((B, 8), f32)

    def p4(st4):
        ib, off = st4
        c0 = pl.multiple_of(ib * B, B)
        krow = keep_ref[0:1, pl.ds(c0, B)]
        prefix = jax.lax.dot_general(krow, ult, (((1,), (0,)), ((), ())),
                                     preferred_element_type=f32)
        cum = off + prefix
        posg = io1 + ib * B
        rnk = jnp.where(krow > 0, cum - 1.0, ktot + posg - cum)
        oh = (rnk == io0).astype(jnp.bfloat16)
        dsb = ds_ref[pl.ds(c0, B), :]
        hi, mid, lo = split3(dsb)
        out_ref[:, :] += (
            jax.lax.dot_general(oh, hi.astype(jnp.bfloat16), dn,
                                preferred_element_type=f32)
            + jax.lax.dot_general(oh, mid.astype(jnp.bfloat16), dn,
                                  preferred_element_type=f32)
            + jax.lax.dot_general(oh, lo.astype(jnp.bfloat16), dn,
                                  preferred_element_type=f32))
        return (ib + 1, off + jnp.sum(krow))

    def p4cond(st4):
        ib, off = st4
        return (ib < NB) & ((off < B8) | (ktot < B8))

    B8 = jnp.float32(128.0)
    jax.lax.while_loop(p4cond, p4, (jnp.int32(0), jnp.float32(0.0)))
    sc = out_ref[:, 4:5]
    out_ref[:, 4:5] = jnp.where(io0 < ktot, sc, -1.0)


@functools.partial(jax.jit, static_argnames=("interpret",))
def _run(draw, interpret=False):
    return pl.pallas_call(
        _body,
        out_shape=jax.ShapeDtypeStruct((B, 8), jnp.float32),
        scratch_shapes=[
            pltpu.VMEM((8, NP), jnp.float32),
            pltpu.VMEM((NP, 8), jnp.float32),
            pltpu.VMEM((8, NP), jnp.float32),
            pltpu.VMEM((1, NP), jnp.float32),
            pltpu.VMEM((NP, 24), jnp.bfloat16),
        ],
        interpret=interpret,
    )(draw)


def kernel(boxes, scores, interpret=False):
    bp = jnp.zeros((NP, 4), jnp.float32).at[:N].set(boxes)
    sp = jnp.zeros((NP, 1), jnp.float32).at[:N, 0].set(scores)
    draw = jnp.concatenate([bp, sp, jnp.zeros((NP, 3), jnp.float32)], axis=1)
    out = _run(draw, interpret=interpret)
    return out[:MAXD, :5]
